# baseline (speedup 1.0000x reference)
import functools

import jax
import jax.numpy as jnp
from jax import lax
from jax.experimental import pallas as pl
from jax.experimental.pallas import tpu as pltpu
from jax.experimental.pallas import tpu_sc as plsc

N = 10000
E = 320000
D_IN = 128
H1 = 96
H2 = 48

NC, NS, L = 2, 16, 16
NW = NC * NS
EPW = E // NW
K = 80
CH = EPW // K
NP = 10240
RPT = NP // NS
ZR = 128
RB = 2000


def _sc_mesh():
    return plsc.VectorSubcoreMesh(core_axis_name="c", subcore_axis_name="s")


def _deg_partials(colf, wf):
    @functools.partial(
        pl.kernel,
        out_type=jax.ShapeDtypeStruct((NC, NP, L), jnp.float32),
        mesh=_sc_mesh(),
        compiler_params=pltpu.CompilerParams(use_tc_tiling_on_sc=False),
        scratch_types=[
            pltpu.VMEM((EPW,), jnp.int32),
            pltpu.VMEM((EPW,), jnp.float32),
            pltpu.VMEM((K, L), jnp.float32),
            pltpu.VMEM((ZR, L), jnp.float32),
            pltpu.VMEM_SHARED((NP, L), jnp.float32),
        ],
    )
    def k(col_hbm, w_hbm, out_hbm, col_v, w_v, wb, zbuf, acc):
        c = lax.axis_index("c")
        s = lax.axis_index("s")
        wid = s * NC + c
        pltpu.sync_copy(col_hbm.at[pl.ds(wid * EPW, EPW)], col_v)
        pltpu.sync_copy(w_hbm.at[pl.ds(wid * EPW, EPW)], w_v)
        zv = jnp.zeros((L,), jnp.float32)

        def zrow(i, carry):
            zbuf[i, :] = zv
            return carry

        lax.fori_loop(0, ZR, zrow, 0)
        for r in range(RPT // ZR):
            pltpu.sync_copy(zbuf, acc.at[pl.ds(s * RPT + r * ZR, ZR)])
        plsc.subcore_barrier()

        def chunk(j, carry):
            def group(gi, cc):
                w16 = w_v[pl.ds(j * K + gi * L, L)]
                for l in range(L):
                    wb[gi * L + l, :] = lax.broadcast_in_dim(
                        lax.slice(w16, (l,), (l + 1,)), (L,), (0,))
                return cc

            lax.fori_loop(0, K // L, group, 0)
            pltpu.sync_copy(wb, acc.at[col_v.at[pl.ds(j * K, K)]], add=True)
            return carry

        lax.fori_loop(0, CH, chunk, 0)
        plsc.subcore_barrier()
        pltpu.sync_copy(acc.at[pl.ds(s * RPT, RPT)],
                        out_hbm.at[c, pl.ds(s * RPT, RPT)])

    return k(colf, wf)


def _agg_partials(g, rowf, colf, wf, h):
    @functools.partial(
        pl.kernel,
        out_type=jax.ShapeDtypeStruct((NC, NP, h), jnp.float32),
        mesh=_sc_mesh(),
        compiler_params=pltpu.CompilerParams(use_tc_tiling_on_sc=False),
        scratch_types=[
            pltpu.VMEM((EPW,), jnp.int32),
            pltpu.VMEM((EPW,), jnp.int32),
            pltpu.VMEM((EPW,), jnp.float32),
            pltpu.VMEM((K, h), jnp.float32),
            pltpu.VMEM((K, h), jnp.float32),
            pltpu.VMEM((K, h), jnp.float32),
            pltpu.VMEM((K, h), jnp.float32),
            pltpu.VMEM_SHARED((NP, h), jnp.float32),
            pltpu.SemaphoreType.DMA,
            pltpu.SemaphoreType.DMA,
            pltpu.SemaphoreType.DMA,
            pltpu.SemaphoreType.DMA,
        ],
    )
    def k(g_hbm, row_hbm, col_hbm, w_hbm, out_hbm,
          row_v, col_v, w_v, ga, gb, oa, ob, acc,
          sga, sgb, ssa, ssb):
        c = lax.axis_index("c")
        s = lax.axis_index("s")
        wid = s * NC + c
        pltpu.sync_copy(row_hbm.at[pl.ds(wid * EPW, EPW)], row_v)
        pltpu.sync_copy(col_hbm.at[pl.ds(wid * EPW, EPW)], col_v)
        pltpu.sync_copy(w_hbm.at[pl.ds(wid * EPW, EPW)], w_v)
        bufs, obufs = (ga, gb), (oa, ob)
        gsems, ssems = (sga, sgb), (ssa, ssb)

        pltpu.async_copy(g_hbm.at[row_v.at[pl.ds(0, K)]], ga, sga)
        zv = jnp.zeros((L,), jnp.float32)

        def zrow(i, carry):
            for f in range(h // L):
                ob[i, pl.ds(f * L, L)] = zv
            return carry

        lax.fori_loop(0, K, zrow, 0)
        for r in range(RPT // K):
            pltpu.sync_copy(ob, acc.at[pl.ds(s * RPT + r * K, K)])
        plsc.subcore_barrier()

        def scale(gbuf, obuf, j):
            def group(gi, cc):
                w16 = w_v[pl.ds(j * K + gi * L, L)]
                for l in range(L):
                    ws = lax.broadcast_in_dim(
                        lax.slice(w16, (l,), (l + 1,)), (L,), (0,))
                    e = gi * L + l
                    for f in range(h // L):
                        sl = pl.ds(f * L, L)
                        obuf[e, sl] = gbuf[e, sl] * ws
                return cc

            lax.fori_loop(0, K // L, group, 0)

        def step(j, b, fire_gather, guard_drain):
            gbuf, obuf = bufs[b], obufs[b]
            pltpu.make_async_copy(g_hbm.at[row_v.at[pl.ds(j * K, K)]], gbuf,
                                  gsems[b]).wait()
            if fire_gather:
                @pl.when(j + 1 < CH)
                def _():
                    pltpu.async_copy(g_hbm.at[row_v.at[pl.ds((j + 1) * K, K)]],
                                     bufs[1 - b], gsems[1 - b])
            scale(gbuf, obuf, j)
            pltpu.async_copy(obuf, acc.at[col_v.at[pl.ds(j * K, K)]],
                             ssems[b], add=True)

            def drain():
                pltpu.make_async_copy(
                    obufs[1 - b], acc.at[col_v.at[pl.ds((j - 1) * K, K)]],
                    ssems[1 - b]).wait()

            if guard_drain:
                pl.when(j >= 1)(drain)
            else:
                drain()

        def pair(i, carry):
            for b in range(2):
                step(2 * i + b, b, fire_gather=True, guard_drain=(b == 0))
            return carry

        lax.fori_loop(0, CH // 2, pair, 0)
        for j in range(2 * (CH // 2), CH):
            step(j, j % 2, fire_gather=False, guard_drain=False)
        bl = (CH - 1) % 2
        pltpu.make_async_copy(obufs[bl], acc.at[col_v.at[pl.ds((CH - 1) * K, K)]],
                              ssems[bl]).wait()
        plsc.subcore_barrier()
        pltpu.sync_copy(acc.at[pl.ds(s * RPT, RPT)],
                        out_hbm.at[c, pl.ds(s * RPT, RPT)])

    return k(g, rowf, colf, wf)


def _dis(degp0, degp1):
    return lax.rsqrt(1.0 + degp0[0, :, :1] + degp1[0, :, :1])


def _sc0(h):
    return pl.BlockSpec((1, RB, h), lambda i: (0, i, 0))


def _sc1(h):
    return pl.BlockSpec((1, RB, h), lambda i: (1, i, 0))


def _tc_g1(x, w1, degp):
    def body(x_ref, w_ref, d0_ref, d1_ref, g_ref):
        dis = _dis(d0_ref, d1_ref)
        g_ref[...] = jnp.dot(x_ref[...], w_ref[...],
                             preferred_element_type=jnp.float32) * dis

    return pl.pallas_call(
        body,
        grid=(N // RB,),
        in_specs=[
            pl.BlockSpec((RB, D_IN), lambda i: (i, 0)),
            pl.BlockSpec((D_IN, H1), lambda i: (0, 0)),
            _sc0(L), _sc1(L),
        ],
        out_specs=pl.BlockSpec((RB, H1), lambda i: (i, 0)),
        out_shape=jax.ShapeDtypeStruct((N, H1), jnp.float32),
    )(x, w1, degp, degp)


def _tc_mid(p, g1, degp, w2, b1):
    def body(p0_ref, p1_ref, g1_ref, d0_ref, d1_ref, w2_ref, b1_ref, out_ref):
        dis = _dis(d0_ref, d1_ref)
        h1 = dis * (p0_ref[0] + p1_ref[0] + g1_ref[...]) + b1_ref[...]
        h1 = jnp.maximum(h1, 0.0)
        out_ref[...] = jnp.dot(h1, w2_ref[...],
                               preferred_element_type=jnp.float32) * dis

    return pl.pallas_call(
        body,
        grid=(N // RB,),
        in_specs=[
            _sc0(H1), _sc1(H1),
            pl.BlockSpec((RB, H1), lambda i: (i, 0)),
            _sc0(L), _sc1(L),
            pl.BlockSpec((H1, H2), lambda i: (0, 0)),
            pl.BlockSpec((1, H1), lambda i: (0, 0)),
        ],
        out_specs=pl.BlockSpec((RB, H2), lambda i: (i, 0)),
        out_shape=jax.ShapeDtypeStruct((N, H2), jnp.float32),
    )(p, p, g1, degp, degp, w2, b1)


def _tc_final(q, g2, degp, wc, b2, bc):
    def body(q0_ref, q1_ref, g2_ref, d0_ref, d1_ref, wc_ref, b2_ref, bc_ref,
             out_ref):
        dis = _dis(d0_ref, d1_ref)
        h2 = dis * (q0_ref[0] + q1_ref[0] + g2_ref[...]) + b2_ref[...]
        h2 = jnp.maximum(h2, 0.0)
        out_ref[...] = jnp.dot(h2, wc_ref[...],
                               preferred_element_type=jnp.float32) + bc_ref[...]

    return pl.pallas_call(
        body,
        grid=(N // RB,),
        in_specs=[
            _sc0(H2), _sc1(H2),
            pl.BlockSpec((RB, H2), lambda i: (i, 0)),
            _sc0(L), _sc1(L),
            pl.BlockSpec((H2, 3), lambda i: (0, 0)),
            pl.BlockSpec((1, H2), lambda i: (0, 0)),
            pl.BlockSpec((1, 3), lambda i: (0, 0)),
        ],
        out_specs=pl.BlockSpec((RB, 3), lambda i: (i, 0)),
        out_shape=jax.ShapeDtypeStruct((N, 3), jnp.float32),
    )(q, q, g2, degp, degp, wc, b2, bc)


def kernel(x, edge_index, edge_weight, W1, b1, W2, b2, Wc, bc):
    row = edge_index[0].astype(jnp.int32)
    col = edge_index[1].astype(jnp.int32)
    w = edge_weight.astype(jnp.float32)

    degp = _deg_partials(col, w)
    g1 = _tc_g1(x, W1, degp)
    p = _agg_partials(g1, row, col, w, H1)
    g2 = _tc_mid(p, g1, degp, W2, b1.reshape(1, H1))
    q = _agg_partials(g2, row, col, w, H2)
    return _tc_final(q, g2, degp, Wc, b2.reshape(1, H2), bc.reshape(1, 3))


# speedup vs baseline: 1.2307x; 1.2307x over previous
import functools

import jax
import jax.numpy as jnp
from jax import lax
from jax.experimental import pallas as pl
from jax.experimental.pallas import tpu as pltpu
from jax.experimental.pallas import tpu_sc as plsc

N = 10000
E = 320000
D_IN = 128
H1 = 96
H2 = 48

NC, NS, L = 2, 16, 16
NW = NC * NS
EPW = E // NW
K = 80
CH = EPW // K
NP = 10240
RPT = NP // NS
ZR = 128
RB = 2000


def _sc_mesh():
    return plsc.VectorSubcoreMesh(core_axis_name="c", subcore_axis_name="s")


def _deg_partials(colf, wf, kc):
    ch = EPW // kc

    @functools.partial(
        pl.kernel,
        out_type=jax.ShapeDtypeStruct((NC, NP, L), jnp.float32),
        mesh=_sc_mesh(),
        compiler_params=pltpu.CompilerParams(use_tc_tiling_on_sc=False),
        scratch_types=[
            pltpu.VMEM((EPW,), jnp.int32),
            pltpu.VMEM((EPW,), jnp.float32),
            pltpu.VMEM((kc, L), jnp.float32),
            pltpu.VMEM((ZR, L), jnp.float32),
            pltpu.VMEM_SHARED((NP, L), jnp.float32),
        ],
    )
    def k(col_hbm, w_hbm, out_hbm, col_v, w_v, wb, zbuf, acc):
        c = lax.axis_index("c")
        s = lax.axis_index("s")
        wid = s * NC + c
        pltpu.sync_copy(col_hbm.at[pl.ds(wid * EPW, EPW)], col_v)
        pltpu.sync_copy(w_hbm.at[pl.ds(wid * EPW, EPW)], w_v)
        zv = jnp.zeros((L,), jnp.float32)

        def zrow(i, carry):
            zbuf[i, :] = zv
            return carry

        lax.fori_loop(0, ZR, zrow, 0)
        for r in range(RPT // ZR):
            pltpu.sync_copy(zbuf, acc.at[pl.ds(s * RPT + r * ZR, ZR)])
        plsc.subcore_barrier()

        def chunk(j, carry):
            def group(gi, cc):
                w16 = w_v[pl.ds(j * kc + gi * L, L)]
                for l in range(L):
                    wb[gi * L + l, :] = lax.broadcast_in_dim(
                        lax.slice(w16, (l,), (l + 1,)), (L,), (0,))
                return cc

            lax.fori_loop(0, kc // L, group, 0)
            pltpu.sync_copy(wb, acc.at[col_v.at[pl.ds(j * kc, kc)]], add=True)
            return carry

        lax.fori_loop(0, ch, chunk, 0)
        plsc.subcore_barrier()
        pltpu.sync_copy(acc.at[pl.ds(s * RPT, RPT)],
                        out_hbm.at[c, pl.ds(s * RPT, RPT)])

    return k(colf, wf)


def _agg_partials(g, rowf, colf, wf, h, kc):
    ch = EPW // kc
    zr = min(kc, ZR)

    @functools.partial(
        pl.kernel,
        out_type=jax.ShapeDtypeStruct((NC, NP, h), jnp.float32),
        mesh=_sc_mesh(),
        compiler_params=pltpu.CompilerParams(use_tc_tiling_on_sc=False),
        scratch_types=[
            pltpu.VMEM((EPW,), jnp.int32),
            pltpu.VMEM((EPW,), jnp.int32),
            pltpu.VMEM((EPW,), jnp.float32),
            pltpu.VMEM((kc, h), jnp.float32),
            pltpu.VMEM((kc, h), jnp.float32),
            pltpu.VMEM((kc, h), jnp.float32),
            pltpu.VMEM_SHARED((NP, h), jnp.float32),
            pltpu.SemaphoreType.DMA,
            pltpu.SemaphoreType.DMA,
            pltpu.SemaphoreType.DMA,
            pltpu.SemaphoreType.DMA,
            pltpu.SemaphoreType.DMA,
            pltpu.SemaphoreType.DMA,
        ],
    )
    def k(g_hbm, row_hbm, col_hbm, w_hbm, out_hbm,
          row_v, col_v, w_v, ga, gb, gc, acc,
          sga, sgb, sgc, ssa, ssb, ssc):
        c = lax.axis_index("c")
        s = lax.axis_index("s")
        wid = s * NC + c
        pltpu.sync_copy(row_hbm.at[pl.ds(wid * EPW, EPW)], row_v)
        pltpu.sync_copy(col_hbm.at[pl.ds(wid * EPW, EPW)], col_v)
        pltpu.sync_copy(w_hbm.at[pl.ds(wid * EPW, EPW)], w_v)
        bufs, gsems, ssems = (ga, gb, gc), (sga, sgb, sgc), (ssa, ssb, ssc)

        pltpu.async_copy(g_hbm.at[row_v.at[pl.ds(0, kc)]], ga, sga)
        pltpu.async_copy(g_hbm.at[row_v.at[pl.ds(kc, kc)]], gb, sgb)
        zv = jnp.zeros((L,), jnp.float32)

        def zrow(i, carry):
            for f in range(h // L):
                gc[i, pl.ds(f * L, L)] = zv
            return carry

        lax.fori_loop(0, zr, zrow, 0)
        for r in range(RPT // zr):
            pltpu.sync_copy(gc.at[pl.ds(0, zr)] if zr < kc else gc,
                            acc.at[pl.ds(s * RPT + r * zr, zr)])
        plsc.subcore_barrier()

        def scale(gbuf, j):
            def group(gi, cc):
                w16 = w_v[pl.ds(j * kc + gi * L, L)]
                for l in range(L):
                    ws = lax.broadcast_in_dim(
                        lax.slice(w16, (l,), (l + 1,)), (L,), (0,))
                    e = gi * L + l
                    for f in range(h // L):
                        sl = pl.ds(f * L, L)
                        gbuf[e, sl] = gbuf[e, sl] * ws
                return cc

            lax.fori_loop(0, kc // L, group, 0)

        def step(j, b, fire_gather, guard_drain):
            gbuf = bufs[b]
            pltpu.make_async_copy(g_hbm.at[row_v.at[pl.ds(j * kc, kc)]], gbuf,
                                  gsems[b]).wait()
            scale(gbuf, j)
            pltpu.async_copy(gbuf, acc.at[col_v.at[pl.ds(j * kc, kc)]],
                             ssems[b], add=True)
            pb = (b + 2) % 3

            def drain():
                pltpu.make_async_copy(
                    bufs[pb], acc.at[col_v.at[pl.ds((j - 1) * kc, kc)]],
                    ssems[pb]).wait()

            if guard_drain:
                pl.when(j >= 1)(drain)
            else:
                drain()
            if fire_gather:
                @pl.when(j + 2 < ch)
                def _():
                    pltpu.async_copy(g_hbm.at[row_v.at[pl.ds((j + 2) * kc, kc)]],
                                     bufs[pb], gsems[pb])

        def triple(i, carry):
            for b in range(3):
                step(3 * i + b, b, fire_gather=True, guard_drain=(b == 0))
            return carry

        lax.fori_loop(0, ch // 3, triple, 0)
        for j in range(3 * (ch // 3), ch):
            step(j, j % 3, fire_gather=False, guard_drain=False)
        bl = (ch - 1) % 3
        pltpu.make_async_copy(bufs[bl], acc.at[col_v.at[pl.ds((ch - 1) * kc, kc)]],
                              ssems[bl]).wait()
        plsc.subcore_barrier()
        pltpu.sync_copy(acc.at[pl.ds(s * RPT, RPT)],
                        out_hbm.at[c, pl.ds(s * RPT, RPT)])

    return k(g, rowf, colf, wf)


def _dis(degp0, degp1):
    return lax.rsqrt(1.0 + degp0[0, :, :1] + degp1[0, :, :1])


def _sc0(h):
    return pl.BlockSpec((1, RB, h), lambda i: (0, i, 0))


def _sc1(h):
    return pl.BlockSpec((1, RB, h), lambda i: (1, i, 0))


def _tc_g1(x, w1, degp):
    def body(x_ref, w_ref, d0_ref, d1_ref, g_ref):
        dis = _dis(d0_ref, d1_ref)
        g_ref[...] = jnp.dot(x_ref[...], w_ref[...],
                             preferred_element_type=jnp.float32) * dis

    return pl.pallas_call(
        body,
        grid=(N // RB,),
        in_specs=[
            pl.BlockSpec((RB, D_IN), lambda i: (i, 0)),
            pl.BlockSpec((D_IN, H1), lambda i: (0, 0)),
            _sc0(L), _sc1(L),
        ],
        out_specs=pl.BlockSpec((RB, H1), lambda i: (i, 0)),
        out_shape=jax.ShapeDtypeStruct((N, H1), jnp.float32),
    )(x, w1, degp, degp)


def _tc_mid(p, g1, degp, w2, b1):
    def body(p0_ref, p1_ref, g1_ref, d0_ref, d1_ref, w2_ref, b1_ref, out_ref):
        dis = _dis(d0_ref, d1_ref)
        h1 = dis * (p0_ref[0] + p1_ref[0] + g1_ref[...]) + b1_ref[...]
        h1 = jnp.maximum(h1, 0.0)
        out_ref[...] = jnp.dot(h1, w2_ref[...],
                               preferred_element_type=jnp.float32) * dis

    return pl.pallas_call(
        body,
        grid=(N // RB,),
        in_specs=[
            _sc0(H1), _sc1(H1),
            pl.BlockSpec((RB, H1), lambda i: (i, 0)),
            _sc0(L), _sc1(L),
            pl.BlockSpec((H1, H2), lambda i: (0, 0)),
            pl.BlockSpec((1, H1), lambda i: (0, 0)),
        ],
        out_specs=pl.BlockSpec((RB, H2), lambda i: (i, 0)),
        out_shape=jax.ShapeDtypeStruct((N, H2), jnp.float32),
    )(p, p, g1, degp, degp, w2, b1)


def _tc_final(q, g2, degp, wc, b2, bc):
    def body(q0_ref, q1_ref, g2_ref, d0_ref, d1_ref, wc_ref, b2_ref, bc_ref,
             out_ref):
        dis = _dis(d0_ref, d1_ref)
        h2 = dis * (q0_ref[0] + q1_ref[0] + g2_ref[...]) + b2_ref[...]
        h2 = jnp.maximum(h2, 0.0)
        out_ref[...] = jnp.dot(h2, wc_ref[...],
                               preferred_element_type=jnp.float32) + bc_ref[...]

    return pl.pallas_call(
        body,
        grid=(N // RB,),
        in_specs=[
            _sc0(H2), _sc1(H2),
            pl.BlockSpec((RB, H2), lambda i: (i, 0)),
            _sc0(L), _sc1(L),
            pl.BlockSpec((H2, 3), lambda i: (0, 0)),
            pl.BlockSpec((1, H2), lambda i: (0, 0)),
            pl.BlockSpec((1, 3), lambda i: (0, 0)),
        ],
        out_specs=pl.BlockSpec((RB, 3), lambda i: (i, 0)),
        out_shape=jax.ShapeDtypeStruct((N, 3), jnp.float32),
    )(q, q, g2, degp, degp, wc, b2, bc)


def kernel(x, edge_index, edge_weight, W1, b1, W2, b2, Wc, bc):
    row = edge_index[0].astype(jnp.int32)
    col = edge_index[1].astype(jnp.int32)
    w = edge_weight.astype(jnp.float32)

    degp = _deg_partials(col, w, K)
    g1 = _tc_g1(x, W1, degp)
    p = _agg_partials(g1, row, col, w, H1, K)
    g2 = _tc_mid(p, g1, degp, W2, b1.reshape(1, H1))
    q = _agg_partials(g2, row, col, w, H2, K)
    return _tc_final(q, g2, degp, Wc, b2.reshape(1, H2), bc.reshape(1, 3))
